# baseline (speedup 1.0000x reference)
import functools

import jax
import jax.numpy as jnp
import numpy as np
from jax import lax
from jax.experimental import pallas as pl
from jax.experimental.pallas import tpu as pltpu
from jax.experimental.pallas import tpu_sc as plsc

_CAT_DIMS = [100000] * 26
_N_FIELDS = 26
_BATCH = 4096
_ED = 16
_OD = 8
_ROWS = _BATCH * _N_FIELDS
_CHUNK = 128
_NW = 32
_NCHUNKS = _ROWS // _CHUNK
_CPW = _NCHUNKS // _NW

_OFFSETS = np.cumsum([0] + _CAT_DIMS[:-1]).astype(np.int32)

_G16 = (np.arange(4096 // 32)[:, None] ==
        (np.arange(4096 // 32 * 16)[None, :] // 16)).astype(np.float32)


def _sc_gather(idx_hbm, etab, ltab, emb_out, lin_out, idx_v, erows, lrows,
               sem_e, sem_l):
  wid = lax.axis_index("s") * 2 + lax.axis_index("c")
  pltpu.sync_copy(idx_hbm.at[wid], idx_v)

  def body(j, carry):
    pltpu.async_copy(etab.at[idx_v.at[j]], erows.at[j], sem_e)
    pltpu.async_copy(ltab.at[idx_v.at[j]], lrows.at[j], sem_l)
    return carry

  lax.fori_loop(0, _CPW, body, 0)
  pltpu.make_async_copy(emb_out.at[wid], erows, sem_e).wait()
  pltpu.make_async_copy(lin_out.at[wid], lrows, sem_l).wait()
  pltpu.sync_copy(erows, emb_out.at[wid])
  pltpu.sync_copy(lrows, lin_out.at[wid])


@functools.lru_cache(maxsize=1)
def _sc_gather_call():
  return pl.kernel(
      _sc_gather,
      out_type=(
          jax.ShapeDtypeStruct((_NW, _CPW, _CHUNK, _ED), jnp.float32),
          jax.ShapeDtypeStruct((_NW, _CPW, _CHUNK, _OD), jnp.float32),
      ),
      mesh=plsc.VectorSubcoreMesh(core_axis_name="c", subcore_axis_name="s"),
      scratch_types=[
          pltpu.VMEM((_CPW, _CHUNK), jnp.int32),
          pltpu.VMEM((_CPW, _CHUNK, _ED), jnp.float32),
          pltpu.VMEM((_CPW, _CHUNK, _OD), jnp.float32),
          pltpu.SemaphoreType.DMA,
          pltpu.SemaphoreType.DMA,
      ],
      compiler_params=pltpu.CompilerParams(use_tc_tiling_on_sc=False),
  )


_NBLK = 32
_BN = _BATCH // _NBLK


def _tc_body(w_ref, i8_ref, g16_ref, e_ref, l_ref, out_ref):
  w = w_ref[...]
  eb = e_ref[...]
  hi = jax.lax.Precision.HIGHEST
  s = lax.dot_general(w, eb, (((1,), (0,)), ((), ())),
                      precision=hi, preferred_element_type=jnp.float32)
  qs = lax.dot_general(w * w, eb * eb, (((1,), (0,)), ((), ())),
                       precision=hi, preferred_element_type=jnp.float32)
  v = 0.5 * (s * s - qs)
  vt = lax.dot_general(v, i8_ref[...], (((0,), (0,)), ((), ())),
                       precision=hi, preferred_element_type=jnp.float32)
  fm = lax.dot_general(g16_ref[...], vt, (((1,), (0,)), ((), ())),
                       precision=hi, preferred_element_type=jnp.float32)
  lin = jnp.sum(l_ref[...], axis=0)
  out_ref[...] = fm + lin


def kernel(cat, lin_table, embed_table, project_weight):
  cat = jnp.asarray(cat, jnp.int32)
  idx = cat + jnp.asarray(_OFFSETS)[None, :]
  idx_t = idx.T.reshape(_NW, _CPW, _CHUNK)

  emb_g, lin_g = _sc_gather_call()(idx_t, embed_table, lin_table)
  e2 = emb_g.reshape(_N_FIELDS, _BATCH * _ED)
  l3 = lin_g.reshape(_N_FIELDS, _BATCH, _OD)
  i8 = jnp.eye(_OD, dtype=jnp.float32)
  g16 = jnp.asarray(_G16)

  out = pl.pallas_call(
      _tc_body,
      grid=(_NBLK,),
      in_specs=[
          pl.BlockSpec((_OD, _N_FIELDS), lambda i: (0, 0)),
          pl.BlockSpec((_OD, _OD), lambda i: (0, 0)),
          pl.BlockSpec((_BN, _BN * _ED), lambda i: (0, 0)),
          pl.BlockSpec((_N_FIELDS, _BN * _ED), lambda i: (0, i)),
          pl.BlockSpec((_N_FIELDS, _BN, _OD), lambda i: (0, i, 0)),
      ],
      out_specs=pl.BlockSpec((_BN, _OD), lambda i: (i, 0)),
      out_shape=jax.ShapeDtypeStruct((_BATCH, _OD), jnp.float32),
  )(project_weight, i8, g16, e2, l3)
  return out
